# baseline (speedup 1.0000x reference)
import functools

import jax
import jax.numpy as jnp
from jax.experimental import pallas as pl
from jax.experimental.pallas import tpu as pltpu

B = 128
S = 16
D_IN = 2048
D_OUT = 4096
E = 8
K = 2
R_DIM = 2432

G = 8
T = (B * K) // G + (E - 1)
SLOTS = T * G
SLOT_PAD = 320
TE_PAD = 64
NT = 1024
NO = D_OUT // NT


def _router_kernel(rf_ref, gw_ref, aux_ref, se_ref, sb_ref, sw_ref):
    rf = rf_ref[...]
    gw = gw_ref[...]
    logits = jax.lax.dot_general(
        rf, gw, (((1,), (1,)), ((), ())),
        precision=jax.lax.Precision.HIGHEST,
        preferred_element_type=jnp.float32)

    eidx = jax.lax.broadcasted_iota(jnp.int32, (B, E), 1)

    l0 = jnp.max(logits, axis=1, keepdims=True)
    a0 = jnp.min(jnp.where(logits == l0, eidx, E), axis=1, keepdims=True)
    oh0 = (eidx == a0)
    masked = jnp.where(oh0, -3e38, logits)
    l1 = jnp.max(masked, axis=1, keepdims=True)
    a1 = jnp.min(jnp.where(masked == l1, eidx, E), axis=1, keepdims=True)
    oh1 = (eidx == a1)
    oh0f = oh0.astype(jnp.float32)
    oh1f = oh1.astype(jnp.float32)

    w1 = 1.0 / (1.0 + jnp.exp(l0 - l1))
    w0 = 1.0 - w1

    ex = jnp.exp(logits - l0)
    probs = ex / jnp.sum(ex, axis=1, keepdims=True)
    pmean = jnp.sum(probs, axis=0, keepdims=True) * (1.0 / B)
    cnt0 = jnp.sum(oh0f, axis=0, keepdims=True)
    cnt1 = jnp.sum(oh1f, axis=0, keepdims=True)
    cnt = cnt0 + cnt1
    frac = cnt * (1.0 / (B * K))
    aux_ref[...] = E * jnp.sum(frac * pmean, axis=1, keepdims=True)

    ri = jax.lax.broadcasted_iota(jnp.int32, (B, B), 0)
    ci = jax.lax.broadcasted_iota(jnp.int32, (B, B), 1)
    tri = (ci < ri).astype(jnp.float32)
    pc0 = jax.lax.dot_general(tri, oh0f, (((1,), (0,)), ((), ())),
                              preferred_element_type=jnp.float32)
    pc1 = jax.lax.dot_general(tri, oh1f, (((1,), (0,)), ((), ())),
                              preferred_element_type=jnp.float32)
    rank0 = jnp.sum(pc0 * oh0f, axis=1, keepdims=True)
    rank1 = (jnp.sum(pc1 * oh1f, axis=1, keepdims=True)
             + jnp.sum(cnt0 * oh1f, axis=1, keepdims=True))

    ntiles = jnp.floor((cnt + (G - 1)) * (1.0 / G))
    ei = jax.lax.broadcasted_iota(jnp.int32, (E, E), 0)
    ej = jax.lax.broadcasted_iota(jnp.int32, (E, E), 1)
    excl = (ei < ej).astype(jnp.float32)
    tbase = jax.lax.dot_general(ntiles, excl, (((1,), (0,)), ((), ())),
                                preferred_element_type=jnp.float32)
    sbase = tbase * G

    slot0 = jnp.sum(sbase * oh0f, axis=1, keepdims=True) + rank0
    slot1 = jnp.sum(sbase * oh1f, axis=1, keepdims=True) + rank1

    sio = jax.lax.broadcasted_iota(jnp.int32, (B, SLOT_PAD), 1).astype(jnp.float32)
    bvec = jax.lax.broadcasted_iota(jnp.int32, (B, 1), 0).astype(jnp.float32)
    m0 = (slot0 == sio).astype(jnp.float32)
    m1 = (slot1 == sio).astype(jnp.float32)
    sb = (jnp.sum(m0 * bvec, axis=0, keepdims=True)
          + jnp.sum(m1 * bvec, axis=0, keepdims=True))
    sw = (jnp.sum(m0 * w0, axis=0, keepdims=True)
          + jnp.sum(m1 * w1, axis=0, keepdims=True))
    sb_ref[...] = sb.astype(jnp.int32)
    sw_ref[...] = sw

    tio = jax.lax.broadcasted_iota(jnp.int32, (TE_PAD, E), 0).astype(jnp.float32)
    owned = (tio >= tbase).astype(jnp.float32)
    se_ref[...] = (jnp.sum(owned, axis=1, keepdims=True) - 1.0).astype(jnp.int32)


def _moe_kernel(se_sm, sb_sm, sw_sm, x_ref, wt_ref, bias_ref, y_ref):
    i = pl.program_id(0)
    t = pl.program_id(1)

    @pl.when(t == 0)
    def _init():
        y_ref[...] = jnp.zeros_like(y_ref)

    wsum = sw_sm[t * G]
    for j in range(1, G):
        wsum = wsum + sw_sm[t * G + j]

    @pl.when(wsum > 0.0)
    def _compute():
        xs = [x_ref[pl.ds(sb_sm[t * G + j] * S, S), :] for j in range(G)]
        xg = jnp.concatenate(xs, axis=0)
        w2 = wt_ref[0]
        acc = jax.lax.dot_general(
            xg, w2, (((1,), (1,)), ((), ())),
            preferred_element_type=jnp.float32)
        et = se_sm[t]
        acc = acc + bias_ref[pl.ds(et, 1), pl.ds(i * NT, NT)]
        for j in range(G):
            bid = sb_sm[t * G + j]
            y_ref[pl.ds(bid * S, S), :] += sw_sm[t * G + j] * acc[j * S:(j + 1) * S, :]


@functools.partial(jax.jit)
def kernel(graph_emb, routing_features, gate_W, expert_W, expert_b):
    aux, se, sb, sw = pl.pallas_call(
        _router_kernel,
        out_shape=(
            jax.ShapeDtypeStruct((1, 1), jnp.float32),
            jax.ShapeDtypeStruct((TE_PAD, 1), jnp.int32),
            jax.ShapeDtypeStruct((1, SLOT_PAD), jnp.int32),
            jax.ShapeDtypeStruct((1, SLOT_PAD), jnp.float32),
        ),
    )(routing_features, gate_W)

    se_arr = se[:T, 0]
    sb_arr = sb[0, :SLOTS]
    sw_arr = sw[0, :SLOTS]
    x = graph_emb.reshape(B * S, D_IN).astype(jnp.bfloat16)
    wt = expert_W.astype(jnp.bfloat16)

    y = pl.pallas_call(
        _moe_kernel,
        grid_spec=pltpu.PrefetchScalarGridSpec(
            num_scalar_prefetch=3,
            grid=(NO, T),
            in_specs=[
                pl.BlockSpec((B * S, D_IN), lambda i, t, *_: (0, 0)),
                pl.BlockSpec((1, NT, D_IN), lambda i, t, se, sb, sw: (se[t], i, 0)),
                pl.BlockSpec((E, D_OUT), lambda i, t, *_: (0, 0)),
            ],
            out_specs=pl.BlockSpec((B * S, NT), lambda i, t, *_: (0, i)),
        ),
        out_shape=jax.ShapeDtypeStruct((B * S, D_OUT), jnp.float32),
        compiler_params=pltpu.CompilerParams(
            dimension_semantics=("arbitrary", "arbitrary"),
        ),
    )(se_arr, sb_arr, sw_arr, x, wt, expert_b)

    return y.reshape(B, S, D_OUT), aux[0, 0]


# speedup vs baseline: 1.4962x; 1.4962x over previous
import functools

import jax
import jax.numpy as jnp
from jax.experimental import pallas as pl
from jax.experimental.pallas import tpu as pltpu

B = 128
S = 16
D_IN = 2048
D_OUT = 4096
E = 8
K = 2
R_DIM = 2432

G = 8
T = (B * K) // G + (E - 1)
SLOTS = T * G
SLOT_PAD = 320
TE_PAD = 64
NT = 1024
NO = D_OUT // NT


NS_PAD = 160
DRING = 2


def _router_kernel(rf_ref, gw_ref, aux_ref, se_ref, sb_ref, sw_ref, sf_ref,
                   chg_ref, new_ref, bid_ref, nc_ref):
    rf = rf_ref[...]
    gw = gw_ref[...]
    logits = jax.lax.dot_general(
        rf, gw, (((1,), (1,)), ((), ())),
        precision=jax.lax.Precision.HIGHEST,
        preferred_element_type=jnp.float32)

    eidx = jax.lax.broadcasted_iota(jnp.int32, (B, E), 1)

    l0 = jnp.max(logits, axis=1, keepdims=True)
    a0 = jnp.min(jnp.where(logits == l0, eidx, E), axis=1, keepdims=True)
    oh0 = (eidx == a0)
    masked = jnp.where(oh0, -3e38, logits)
    l1 = jnp.max(masked, axis=1, keepdims=True)
    a1 = jnp.min(jnp.where(masked == l1, eidx, E), axis=1, keepdims=True)
    oh1 = (eidx == a1)
    oh0f = oh0.astype(jnp.float32)
    oh1f = oh1.astype(jnp.float32)

    w1 = 1.0 / (1.0 + jnp.exp(l0 - l1))
    w0 = 1.0 - w1

    ex = jnp.exp(logits - l0)
    probs = ex / jnp.sum(ex, axis=1, keepdims=True)
    pmean = jnp.sum(probs, axis=0, keepdims=True) * (1.0 / B)
    cnt0 = jnp.sum(oh0f, axis=0, keepdims=True)
    cnt1 = jnp.sum(oh1f, axis=0, keepdims=True)
    cnt = cnt0 + cnt1
    frac = cnt * (1.0 / (B * K))
    aux_ref[...] = E * jnp.sum(frac * pmean, axis=1, keepdims=True)

    ri = jax.lax.broadcasted_iota(jnp.int32, (B, B), 0)
    ci = jax.lax.broadcasted_iota(jnp.int32, (B, B), 1)
    tri = (ci < ri).astype(jnp.float32)
    pc0 = jax.lax.dot_general(tri, oh0f, (((1,), (0,)), ((), ())),
                              preferred_element_type=jnp.float32)
    pc1 = jax.lax.dot_general(tri, oh1f, (((1,), (0,)), ((), ())),
                              preferred_element_type=jnp.float32)
    rank0 = jnp.sum(pc0 * oh0f, axis=1, keepdims=True)
    rank1 = (jnp.sum(pc1 * oh1f, axis=1, keepdims=True)
             + jnp.sum(cnt0 * oh1f, axis=1, keepdims=True))

    ntiles = jnp.floor((cnt + (G - 1)) * (1.0 / G))
    ei = jax.lax.broadcasted_iota(jnp.int32, (E, E), 0)
    ej = jax.lax.broadcasted_iota(jnp.int32, (E, E), 1)
    excl = (ei < ej).astype(jnp.float32)
    tbase = jax.lax.dot_general(ntiles, excl, (((1,), (0,)), ((), ())),
                                preferred_element_type=jnp.float32)
    sbase = tbase * G

    slot0 = jnp.sum(sbase * oh0f, axis=1, keepdims=True) + rank0
    slot1 = jnp.sum(sbase * oh1f, axis=1, keepdims=True) + rank1
    first0 = (slot0 < slot1).astype(jnp.float32)
    first1 = 1.0 - first0

    sio = jax.lax.broadcasted_iota(jnp.int32, (B, SLOT_PAD), 1).astype(jnp.float32)
    bvec = jax.lax.broadcasted_iota(jnp.int32, (B, 1), 0).astype(jnp.float32)
    m0 = (slot0 == sio).astype(jnp.float32)
    m1 = (slot1 == sio).astype(jnp.float32)
    sb = (jnp.sum(m0 * bvec, axis=0, keepdims=True)
          + jnp.sum(m1 * bvec, axis=0, keepdims=True))
    sw = (jnp.sum(m0 * w0, axis=0, keepdims=True)
          + jnp.sum(m1 * w1, axis=0, keepdims=True))
    sf = (jnp.sum(m0 * first0, axis=0, keepdims=True)
          + jnp.sum(m1 * first1, axis=0, keepdims=True))
    sb_ref[...] = sb.astype(jnp.int32)
    sw_ref[...] = sw
    sf_ref[...] = sf.astype(jnp.int32)

    tio = jax.lax.broadcasted_iota(jnp.int32, (TE_PAD, E), 0).astype(jnp.float32)
    owned = (tio >= tbase).astype(jnp.float32)
    eotf = jnp.sum(owned, axis=1, keepdims=True) - 1.0
    se_ref[...] = eotf.astype(jnp.int32)

    prev = jnp.concatenate([eotf[0:1] + 1.0, eotf[:-1]], axis=0)
    nrt = (eotf != prev).astype(jnp.float32)
    sio2 = jax.lax.broadcasted_iota(jnp.int32, (NS_PAD, 1), 0).astype(jnp.float32)
    kT = (jax.lax.broadcasted_iota(jnp.int32, (1, NO), 1).astype(jnp.float32) * T)
    iic = jnp.sum((sio2 >= kT).astype(jnp.float32), axis=1, keepdims=True) - 1.0
    tic = sio2 - iic * T
    tcol = jax.lax.broadcasted_iota(jnp.int32, (1, TE_PAD), 1).astype(jnp.float32)
    oht = (tic == tcol).astype(jnp.float32)
    se_step = jax.lax.dot_general(oht, eotf, (((1,), (0,)), ((), ())),
                                  preferred_element_type=jnp.float32)
    nr_step = jax.lax.dot_general(oht, nrt, (((1,), (0,)), ((), ())),
                                  preferred_element_type=jnp.float32)
    valid = (sio2 < (NO * T)).astype(jnp.float32)
    newv = nr_step * valid
    si = jax.lax.broadcasted_iota(jnp.int32, (NS_PAD, NS_PAD), 0)
    sj = jax.lax.broadcasted_iota(jnp.int32, (NS_PAD, NS_PAD), 1)
    tris = (sj <= si).astype(jnp.float32)
    cstep = jax.lax.dot_general(tris, newv, (((1,), (0,)), ((), ())),
                                preferred_element_type=jnp.float32) - 1.0
    bstep = se_step * NO + iic
    kcol = jax.lax.broadcasted_iota(jnp.int32, (1, TE_PAD), 1).astype(jnp.float32)
    ohc = (cstep == kcol).astype(jnp.float32) * newv
    bidc = jax.lax.dot_general(ohc, bstep, (((0,), (0,)), ((), ())),
                               preferred_element_type=jnp.float32)
    chg_ref[...] = cstep.astype(jnp.int32)
    new_ref[...] = newv.astype(jnp.int32)
    bid_ref[...] = bidc.astype(jnp.int32)
    nc_ref[...] = jnp.sum(newv, axis=0, keepdims=True).astype(jnp.int32)


def _moe_kernel(se_sm, sb_sm, sw_sm, chg_sm, new_sm, bid_sm, nc_sm,
                x_ref, wr_ref, bias_ref, y_ref, ring_ref, sems, f_ref):
    i = pl.program_id(0)
    t = pl.program_id(1)
    s = i * T + t

    @pl.when(s == 0)
    def _init_f():
        f_ref[0] = 0

    c = chg_sm[s]
    nc = nc_sm[0]
    for _ in range(2):
        f = f_ref[0]

        @pl.when((f < nc) & (f < c + DRING))
        def _issue(f=f):
            bid = bid_sm[f]
            slot = jax.lax.rem(f, DRING)
            pltpu.make_async_copy(
                wr_ref.at[pl.ds(bid * NT, NT), :],
                ring_ref.at[pl.ds(slot * NT, NT), :],
                sems.at[slot],
            ).start()
            f_ref[0] = f + 1

    @pl.when(new_sm[s] == 1)
    def _wait():
        slot = jax.lax.rem(c, DRING)
        pltpu.make_async_copy(
            wr_ref.at[pl.ds(0, NT), :],
            ring_ref.at[pl.ds(slot * NT, NT), :],
            sems.at[slot],
        ).wait()

    @pl.when(t == 0)
    def _init():
        y_ref[...] = jnp.zeros_like(y_ref)

    wsum = sw_sm[t * G]
    for j in range(1, G):
        wsum = wsum + sw_sm[t * G + j]

    @pl.when(wsum > 0.0)
    def _compute():
        slot = jax.lax.rem(c, DRING)
        w = ring_ref[pl.ds(slot * NT, NT), :]
        xs = [x_ref[pl.ds(sb_sm[t * G + j] * S, S), :] for j in range(G)]
        xg = jnp.concatenate(xs, axis=0)
        acc = jax.lax.dot_general(
            xg, w, (((1,), (1,)), ((), ())),
            preferred_element_type=jnp.float32)
        et = se_sm[t]
        acc = acc + bias_ref[pl.ds(et, 1), pl.ds(i * NT, NT)]
        for j in range(G):
            bid = sb_sm[t * G + j]
            y_ref[pl.ds(bid * S, S), :] += sw_sm[t * G + j] * acc[j * S:(j + 1) * S, :]


@functools.partial(jax.jit)
def kernel(graph_emb, routing_features, gate_W, expert_W, expert_b):
    aux, se, sb, sw, sf, chg, new, bid, nc = pl.pallas_call(
        _router_kernel,
        out_shape=(
            jax.ShapeDtypeStruct((1, 1), jnp.float32),
            jax.ShapeDtypeStruct((TE_PAD, 1), jnp.int32),
            jax.ShapeDtypeStruct((1, SLOT_PAD), jnp.int32),
            jax.ShapeDtypeStruct((1, SLOT_PAD), jnp.float32),
            jax.ShapeDtypeStruct((1, SLOT_PAD), jnp.int32),
            jax.ShapeDtypeStruct((NS_PAD, 1), jnp.int32),
            jax.ShapeDtypeStruct((NS_PAD, 1), jnp.int32),
            jax.ShapeDtypeStruct((TE_PAD, 1), jnp.int32),
            jax.ShapeDtypeStruct((1, 1), jnp.int32),
        ),
    )(routing_features, gate_W)

    se_arr = se[:T, 0]
    sb_arr = sb[0, :SLOTS]
    sw_arr = sw[0, :SLOTS]
    chg_arr = chg[:, 0]
    new_arr = new[:, 0]
    bid_arr = bid[:, 0]
    nc_arr = nc[0]
    x = graph_emb.reshape(B * S, D_IN)
    wr = expert_W.reshape(E * D_OUT, D_IN)

    y = pl.pallas_call(
        _moe_kernel,
        grid_spec=pltpu.PrefetchScalarGridSpec(
            num_scalar_prefetch=7,
            grid=(NO, T),
            in_specs=[
                pl.BlockSpec((B * S, D_IN), lambda i, t, *_: (0, 0)),
                pl.BlockSpec(memory_space=pl.ANY),
                pl.BlockSpec((E, D_OUT), lambda i, t, *_: (0, 0)),
            ],
            out_specs=pl.BlockSpec((B * S, NT), lambda i, t, *_: (0, i)),
            scratch_shapes=[
                pltpu.VMEM((DRING * NT, D_IN), jnp.float32),
                pltpu.SemaphoreType.DMA((DRING,)),
                pltpu.SMEM((1,), jnp.int32),
            ],
        ),
        out_shape=jax.ShapeDtypeStruct((B * S, D_OUT), jnp.float32),
        compiler_params=pltpu.CompilerParams(
            dimension_semantics=("arbitrary", "arbitrary"),
        ),
    )(se_arr, sb_arr, sw_arr, chg_arr, new_arr, bid_arr, nc_arr,
      x, wr, expert_b)

    return y.reshape(B, S, D_OUT), aux[0, 0]
